# Initial kernel scaffold; baseline (speedup 1.0000x reference)
import jax
import jax.numpy as jnp
from jax.experimental import pallas as pl


def kernel(logits, label):
    raise NotImplementedError("write your pallas kernel here")


# speedup vs baseline: 145.3998x; 145.3998x over previous
import functools

import jax
import jax.numpy as jnp
from jax import lax
from jax.experimental import pallas as pl
from jax.experimental.pallas import tpu as pltpu
from jax.experimental.pallas import tpu_sc as plsc

_NB = 2048
_NB2 = 2 * _NB
_U = 12.0
_SCALE = _NB / (2.0 * _U)
_N = 8 * 512 * 512
_NC = 2
_NS = 16
_NW = _NC * _NS
_PER = _N // _NW
_CHUNK = 8192
_NCH = _PER // _CHUNK
_LANES = 16
_HWORDS = _LANES * _NB2


def _sc_body(x_hbm, lb_hbm, out_hbm, hist, xbuf, lbuf, stage, sem0, sem1):
    wid = lax.axis_index("s") * _NC + lax.axis_index("c")
    base = wid * _PER
    sems = (sem0, sem1)

    def issue(g):
        slot = g % 2
        off = base + g * _CHUNK
        hx = pltpu.async_copy(x_hbm.at[pl.ds(off, _CHUNK)], xbuf.at[slot],
                              sems[slot])
        hl = pltpu.async_copy(lb_hbm.at[pl.ds(off, _CHUNK)], lbuf.at[slot],
                              sems[slot])
        return (hx, hl)

    pending = issue(0)

    zeros16 = jnp.zeros((_LANES,), jnp.float32)

    def zbody(j, carry):
        hist[pl.ds(j * _LANES, _LANES)] = zeros16
        return carry

    lax.fori_loop(0, _HWORDS // _LANES, zbody, 0)

    lane_off = lax.iota(jnp.int32, _LANES) * _NB2
    ones16 = jnp.ones((_LANES,), jnp.float32)

    for g in range(_NCH):
        nxt = issue(g + 1) if g + 1 < _NCH else None
        pending[0].wait()
        pending[1].wait()
        slot = g % 2

        def pbody(i, carry):
            x = xbuf[slot, pl.ds(i * _LANES, _LANES)]
            lb = lbuf[slot, pl.ds(i * _LANES, _LANES)]
            s = (x + x) - 1.0
            u = jnp.where(lb > 0, -s, s)
            v = (u + _U) * _SCALE
            bi = jnp.minimum(jnp.maximum(v.astype(jnp.int32), 0), _NB - 1)
            bdesc = (_NB - 1) - bi
            idx = lb * _NB + bdesc + lane_off
            plsc.addupdate_scatter(hist, [idx], ones16)
            return carry

        lax.fori_loop(0, _CHUNK // _LANES, pbody, 0)
        if nxt is not None:
            pending = nxt

    def rbody(j, carry):
        acc = hist[pl.ds(j * _LANES, _LANES)]
        for l in range(1, _LANES):
            acc = acc + hist[pl.ds(l * _NB2 + j * _LANES, _LANES)]
        stage[pl.ds(j * _LANES, _LANES)] = acc
        return carry

    lax.fori_loop(0, _NB2 // _LANES, rbody, 0)
    pltpu.sync_copy(stage, out_hbm.at[wid])


_sc_hist = functools.partial(
    pl.kernel,
    out_type=jax.ShapeDtypeStruct((_NW, _NB2), jnp.float32),
    mesh=plsc.VectorSubcoreMesh(core_axis_name="c", subcore_axis_name="s"),
    scratch_types=[
        pltpu.VMEM((_HWORDS,), jnp.float32),
        pltpu.VMEM((2, _CHUNK), jnp.float32),
        pltpu.VMEM((2, _CHUNK), jnp.int32),
        pltpu.VMEM((_NB2,), jnp.float32),
        pltpu.SemaphoreType.DMA,
        pltpu.SemaphoreType.DMA,
    ],
    compiler_params=pltpu.CompilerParams(needs_layout_passes=False),
)(_sc_body)


def _tc_finish(h_ref, o_ref):
    h = h_ref[...]
    cnt = jnp.sum(h, axis=0)
    c0 = cnt[0]
    c1 = cnt[1]

    i128r = lax.broadcasted_iota(jnp.int32, (128, 128), 0)
    i128c = lax.broadcasted_iota(jnp.int32, (128, 128), 1)
    umat = (i128r <= i128c).astype(jnp.float32)
    i16r = lax.broadcasted_iota(jnp.int32, (16, 16), 0)
    i16c = lax.broadcasted_iota(jnp.int32, (16, 16), 1)
    lmat = (i16c < i16r).astype(jnp.float32)

    def incl_cumsum(c):
        srow = lax.dot_general(c, umat, (((1,), (0,)), ((), ())),
                               precision=lax.Precision.HIGHEST,
                               preferred_element_type=jnp.float32)
        rowsum = jnp.sum(c, axis=1, keepdims=True)
        offs = lax.dot_general(lmat, rowsum, (((1,), (0,)), ((), ())),
                               precision=lax.Precision.HIGHEST,
                               preferred_element_type=jnp.float32)
        return srow + offs

    s0 = incl_cumsum(c0)
    s1 = incl_cumsum(c1)
    t = s0 + s1
    n0 = jnp.sum(c0)
    n1 = jnp.sum(c1)

    def jacc(nc, sc, so):
        return jnp.where(t == 0.0, 0.0,
                         1.0 - (nc - sc) / jnp.maximum(nc + so, 1.0))

    j0 = jacc(n0, s0, s1)
    j1 = jacc(n1, s1, s0)

    k = (lax.broadcasted_iota(jnp.int32, (16, 128), 0) * 128
         + lax.broadcasted_iota(jnp.int32, (16, 128), 1)).astype(jnp.float32)
    du = 2.0 * _U / _NB
    ucent = _U - (k + 0.5) * du
    ebar = 1.0 / (1.0 + jnp.exp(-ucent))
    ebar_next = 1.0 / (1.0 + jnp.exp(-(ucent - du)))
    w = ebar - jnp.where(k == float(_NB - 1), 0.0, ebar_next)

    loss = 0.5 * (jnp.sum(w * j0) + jnp.sum(w * j1))
    o_ref[...] = jnp.broadcast_to(loss, (1, 1))


def kernel(logits, label):
    xf = logits.reshape(-1)
    lb = label.astype(jnp.int32).reshape(-1)
    part = _sc_hist(xf, lb)
    part4 = part.reshape(_NW, 2, 16, 128)
    out = pl.pallas_call(
        _tc_finish,
        out_shape=jax.ShapeDtypeStruct((1, 1), jnp.float32),
    )(part4)
    return out[0, 0]


# speedup vs baseline: 315.2621x; 2.1682x over previous
import functools

import jax
import jax.numpy as jnp
from jax import lax
from jax.experimental import pallas as pl
from jax.experimental.pallas import tpu as pltpu
from jax.experimental.pallas import tpu_sc as plsc

_NB = 2048
_NB2 = 2 * _NB
_U = 12.0
_SCALE = _NB / (2.0 * _U)
_N = 8 * 512 * 512
_NC = 2
_NS = 16
_NW = _NC * _NS
_PER = _N // _NW
_CHUNK = 8192
_NCH = _PER // _CHUNK
_LANES = 16
_HWORDS = _LANES * _NB2


def _sc_body(x_hbm, lb_hbm, out_hbm, hist, xbuf, lbuf, stage, sem0, sem1):
    wid = lax.axis_index("s") * _NC + lax.axis_index("c")
    base = wid * _PER
    sems = (sem0, sem1)

    def issue(g):
        slot = g % 2
        off = base + g * _CHUNK
        hx = pltpu.async_copy(x_hbm.at[pl.ds(off, _CHUNK)], xbuf.at[slot],
                              sems[slot])
        hl = pltpu.async_copy(lb_hbm.at[pl.ds(off, _CHUNK)], lbuf.at[slot],
                              sems[slot])
        return (hx, hl)

    pending = issue(0)

    zeros16 = jnp.zeros((_LANES,), jnp.float32)

    @plsc.parallel_loop(0, _HWORDS, step=_LANES, unroll=8)
    def _zero(j):
        hist[pl.ds(j, _LANES)] = zeros16

    lane_off = lax.iota(jnp.int32, _LANES) * _NB2
    ones16 = jnp.ones((_LANES,), jnp.float32)

    for g in range(_NCH):
        nxt = issue(g + 1) if g + 1 < _NCH else None
        pending[0].wait()
        pending[1].wait()
        slot = g % 2

        @plsc.parallel_loop(0, _CHUNK, step=_LANES, unroll=4)
        def _accum(i):
            x = xbuf[slot, pl.ds(i, _LANES)]
            lb = lbuf[slot, pl.ds(i, _LANES)]
            s = (x + x) - 1.0
            u = jnp.where(lb > 0, -s, s)
            v = (u + _U) * _SCALE
            bi = jnp.minimum(jnp.maximum(v.astype(jnp.int32), 0), _NB - 1)
            bdesc = (_NB - 1) - bi
            idx = lb * _NB + bdesc + lane_off
            plsc.addupdate_scatter(hist, [idx], ones16)
        if nxt is not None:
            pending = nxt

    @plsc.parallel_loop(0, _NB2, step=_LANES, unroll=2)
    def _reduce(j):
        acc = hist[pl.ds(j, _LANES)]
        for l in range(1, _LANES):
            acc = acc + hist[pl.ds(l * _NB2 + j, _LANES)]
        stage[pl.ds(j, _LANES)] = acc
    pltpu.sync_copy(stage, out_hbm.at[wid])


_sc_hist = functools.partial(
    pl.kernel,
    out_type=jax.ShapeDtypeStruct((_NW, _NB2), jnp.float32),
    mesh=plsc.VectorSubcoreMesh(core_axis_name="c", subcore_axis_name="s"),
    scratch_types=[
        pltpu.VMEM((_HWORDS,), jnp.float32),
        pltpu.VMEM((2, _CHUNK), jnp.float32),
        pltpu.VMEM((2, _CHUNK), jnp.int32),
        pltpu.VMEM((_NB2,), jnp.float32),
        pltpu.SemaphoreType.DMA,
        pltpu.SemaphoreType.DMA,
    ],
    compiler_params=pltpu.CompilerParams(needs_layout_passes=False),
)(_sc_body)


def _tc_finish(h_ref, o_ref):
    h = h_ref[...]
    cnt = jnp.sum(h, axis=0)
    c0 = cnt[0]
    c1 = cnt[1]

    i128r = lax.broadcasted_iota(jnp.int32, (128, 128), 0)
    i128c = lax.broadcasted_iota(jnp.int32, (128, 128), 1)
    umat = (i128r <= i128c).astype(jnp.float32)
    i16r = lax.broadcasted_iota(jnp.int32, (16, 16), 0)
    i16c = lax.broadcasted_iota(jnp.int32, (16, 16), 1)
    lmat = (i16c < i16r).astype(jnp.float32)

    def incl_cumsum(c):
        srow = lax.dot_general(c, umat, (((1,), (0,)), ((), ())),
                               precision=lax.Precision.HIGHEST,
                               preferred_element_type=jnp.float32)
        rowsum = jnp.sum(c, axis=1, keepdims=True)
        offs = lax.dot_general(lmat, rowsum, (((1,), (0,)), ((), ())),
                               precision=lax.Precision.HIGHEST,
                               preferred_element_type=jnp.float32)
        return srow + offs

    s0 = incl_cumsum(c0)
    s1 = incl_cumsum(c1)
    t = s0 + s1
    n0 = jnp.sum(c0)
    n1 = jnp.sum(c1)

    def jacc(nc, sc, so):
        return jnp.where(t == 0.0, 0.0,
                         1.0 - (nc - sc) / jnp.maximum(nc + so, 1.0))

    j0 = jacc(n0, s0, s1)
    j1 = jacc(n1, s1, s0)

    k = (lax.broadcasted_iota(jnp.int32, (16, 128), 0) * 128
         + lax.broadcasted_iota(jnp.int32, (16, 128), 1)).astype(jnp.float32)
    du = 2.0 * _U / _NB
    ucent = _U - (k + 0.5) * du
    ebar = 1.0 / (1.0 + jnp.exp(-ucent))
    ebar_next = 1.0 / (1.0 + jnp.exp(-(ucent - du)))
    w = ebar - jnp.where(k == float(_NB - 1), 0.0, ebar_next)

    loss = 0.5 * (jnp.sum(w * j0) + jnp.sum(w * j1))
    o_ref[...] = jnp.broadcast_to(loss, (1, 1))


def kernel(logits, label):
    xf = logits.reshape(-1)
    lb = label.astype(jnp.int32).reshape(-1)
    part = _sc_hist(xf, lb)
    part4 = part.reshape(_NW, 2, 16, 128)
    out = pl.pallas_call(
        _tc_finish,
        out_shape=jax.ShapeDtypeStruct((1, 1), jnp.float32),
    )(part4)
    return out[0, 0]


# speedup vs baseline: 336.9599x; 1.0688x over previous
import functools

import jax
import jax.numpy as jnp
from jax import lax
from jax.experimental import pallas as pl
from jax.experimental.pallas import tpu as pltpu
from jax.experimental.pallas import tpu_sc as plsc

_NB = 2048
_NB2 = 2 * _NB
_U = 12.0
_SCALE = _NB / (2.0 * _U)
_N = 8 * 512 * 512
_NC = 2
_NS = 16
_NW = _NC * _NS
_PER = _N // _NW
_CHUNK = 8192
_NCH = _PER // _CHUNK
_LANES = 16
_HWORDS = _LANES * _NB2


def _sc_body(x_hbm, lb_hbm, out_hbm, hist, xbuf, lbuf, stage, sem0, sem1):
    wid = lax.axis_index("s") * _NC + lax.axis_index("c")
    base = wid * _PER
    sems = (sem0, sem1)

    def issue(g):
        slot = g % 2
        off = base + g * _CHUNK
        hx = pltpu.async_copy(x_hbm.at[pl.ds(off, _CHUNK)], xbuf.at[slot],
                              sems[slot])
        hl = pltpu.async_copy(lb_hbm.at[pl.ds(off, _CHUNK)], lbuf.at[slot],
                              sems[slot])
        return (hx, hl)

    pending = issue(0)

    zeros16 = jnp.zeros((_LANES,), jnp.float32)

    @plsc.parallel_loop(0, _HWORDS, step=_LANES, unroll=8)
    def _zero(j):
        hist[pl.ds(j, _LANES)] = zeros16

    lane_off = lax.iota(jnp.int32, _LANES) * _NB2
    ones16 = jnp.ones((_LANES,), jnp.float32)
    a0 = 2.0 * _SCALE
    b0 = (_U - 1.0) * _SCALE
    b1 = (_U + 1.0) * _SCALE
    base0 = lane_off + (_NB - 1)
    base1 = lane_off + (_NB2 - 1)
    vmaxf = float(_NB) - 0.5

    for g in range(_NCH):
        nxt = issue(g + 1) if g + 1 < _NCH else None
        pending[0].wait()
        pending[1].wait()
        slot = g % 2

        @plsc.parallel_loop(0, _CHUNK, step=_LANES, unroll=8)
        def _accum(i):
            x = xbuf[slot, pl.ds(i, _LANES)]
            lb = lbuf[slot, pl.ds(i, _LANES)]
            pos = lb > 0
            a = jnp.where(pos, -a0, a0)
            b = jnp.where(pos, b1, b0)
            v = a * x + b
            v = jnp.minimum(jnp.maximum(v, 0.0), vmaxf)
            bi = v.astype(jnp.int32)
            base = jnp.where(pos, base1, base0)
            idx = base - bi
            plsc.addupdate_scatter(hist, [idx], ones16)
        if nxt is not None:
            pending = nxt

    @plsc.parallel_loop(0, _NB2, step=_LANES, unroll=2)
    def _reduce(j):
        acc = hist[pl.ds(j, _LANES)]
        for l in range(1, _LANES):
            acc = acc + hist[pl.ds(l * _NB2 + j, _LANES)]
        stage[pl.ds(j, _LANES)] = acc
    pltpu.sync_copy(stage, out_hbm.at[wid])


_sc_hist = functools.partial(
    pl.kernel,
    out_type=jax.ShapeDtypeStruct((_NW, _NB2), jnp.float32),
    mesh=plsc.VectorSubcoreMesh(core_axis_name="c", subcore_axis_name="s"),
    scratch_types=[
        pltpu.VMEM((_HWORDS,), jnp.float32),
        pltpu.VMEM((2, _CHUNK), jnp.float32),
        pltpu.VMEM((2, _CHUNK), jnp.int32),
        pltpu.VMEM((_NB2,), jnp.float32),
        pltpu.SemaphoreType.DMA,
        pltpu.SemaphoreType.DMA,
    ],
    compiler_params=pltpu.CompilerParams(needs_layout_passes=False),
)(_sc_body)


def _tc_finish(h_ref, o_ref):
    h = h_ref[...]
    cnt = jnp.sum(h, axis=0)
    c0 = cnt[0]
    c1 = cnt[1]

    i128r = lax.broadcasted_iota(jnp.int32, (128, 128), 0)
    i128c = lax.broadcasted_iota(jnp.int32, (128, 128), 1)
    umat = (i128r <= i128c).astype(jnp.float32)
    i16r = lax.broadcasted_iota(jnp.int32, (16, 16), 0)
    i16c = lax.broadcasted_iota(jnp.int32, (16, 16), 1)
    lmat = (i16c < i16r).astype(jnp.float32)

    def incl_cumsum(c):
        srow = lax.dot_general(c, umat, (((1,), (0,)), ((), ())),
                               precision=lax.Precision.HIGHEST,
                               preferred_element_type=jnp.float32)
        rowsum = jnp.sum(c, axis=1, keepdims=True)
        offs = lax.dot_general(lmat, rowsum, (((1,), (0,)), ((), ())),
                               precision=lax.Precision.HIGHEST,
                               preferred_element_type=jnp.float32)
        return srow + offs

    s0 = incl_cumsum(c0)
    s1 = incl_cumsum(c1)
    t = s0 + s1
    n0 = jnp.sum(c0)
    n1 = jnp.sum(c1)

    def jacc(nc, sc, so):
        return jnp.where(t == 0.0, 0.0,
                         1.0 - (nc - sc) / jnp.maximum(nc + so, 1.0))

    j0 = jacc(n0, s0, s1)
    j1 = jacc(n1, s1, s0)

    k = (lax.broadcasted_iota(jnp.int32, (16, 128), 0) * 128
         + lax.broadcasted_iota(jnp.int32, (16, 128), 1)).astype(jnp.float32)
    du = 2.0 * _U / _NB
    ucent = _U - (k + 0.5) * du
    ebar = 1.0 / (1.0 + jnp.exp(-ucent))
    ebar_next = 1.0 / (1.0 + jnp.exp(-(ucent - du)))
    w = ebar - jnp.where(k == float(_NB - 1), 0.0, ebar_next)

    loss = 0.5 * (jnp.sum(w * j0) + jnp.sum(w * j1))
    o_ref[...] = jnp.broadcast_to(loss, (1, 1))


def kernel(logits, label):
    xf = logits.reshape(-1)
    lb = label.astype(jnp.int32).reshape(-1)
    part = _sc_hist(xf, lb)
    part4 = part.reshape(_NW, 2, 16, 128)
    out = pl.pallas_call(
        _tc_finish,
        out_shape=jax.ShapeDtypeStruct((1, 1), jnp.float32),
    )(part4)
    return out[0, 0]
